# Initial kernel scaffold; baseline (speedup 1.0000x reference)
import jax
import jax.numpy as jnp
from jax.experimental import pallas as pl


def kernel(x, keys, W, b):
    raise NotImplementedError("write your pallas kernel here")


# speedup vs baseline: 2.0782x; 2.0782x over previous
import jax
import jax.numpy as jnp
from jax.experimental import pallas as pl

M = 64
C = 64
D = 1024
K = 8
TF = 10.0
BB = 512


def _fused_kernel(x_ref, keys_ref, w_ref, b_ref, out_ref):
    xb = x_ref[...]
    nrm = jnp.sqrt(jnp.sum(xb * xb, axis=1, keepdims=True))
    xn = xb / jnp.maximum(nrm, 1e-12)
    kraw = keys_ref[...]
    knrm = jnp.sqrt(jnp.sum(kraw * kraw, axis=1, keepdims=True))
    kn = kraw / jnp.maximum(knrm, 1e-12)
    cos = jax.lax.dot_general(
        xn, kn, (((1,), (1,)), ((), ())),
        preferred_element_type=jnp.float32)

    iota = jax.lax.broadcasted_iota(jnp.int32, (BB, M), 1)
    work = cos
    mask = jnp.zeros((BB, M), jnp.bool_)
    for _ in range(K):
        mx = jnp.max(work, axis=1, keepdims=True)
        is_mx = work == mx
        first = jnp.min(jnp.where(is_mx, iota, M), axis=1, keepdims=True)
        sel = iota == first
        mask = jnp.logical_or(mask, sel)
        work = jnp.where(sel, -jnp.inf, work)

    gate = jnp.where(mask, cos, 0.0)
    den = jnp.sum(gate, axis=1, keepdims=True)

    xb16 = xb.astype(jnp.bfloat16)
    lane = jax.lax.broadcasted_iota(jnp.int32, (BB, 2 * C), 1)
    acc = jnp.zeros((BB, 2 * C), jnp.float32)
    for mp in range(M // 2):
        wpair = w_ref[pl.ds(mp * 2 * C, 2 * C), :]
        raw = jax.lax.dot_general(
            xb16, wpair, (((1,), (1,)), ((), ())),
            preferred_element_type=jnp.float32)
        raw = raw + b_ref[0, pl.ds(mp * 2 * C, 2 * C)][None, :]
        t = jnp.tanh(raw * (1.0 / TF)) * TF
        w0 = gate[:, 2 * mp][:, None]
        w1 = gate[:, 2 * mp + 1][:, None]
        wvec = jnp.where(lane < C, w0, w1)
        acc = acc + t * wvec
    out_ref[...] = (acc[:, :C] + acc[:, C:]) / den


def kernel(x, keys, W, b):
    B = x.shape[0]
    Wf = W.reshape(M * C, D).astype(jnp.bfloat16)
    bf = b.reshape(1, M * C)
    grid = (B // BB,)
    return pl.pallas_call(
        _fused_kernel,
        grid=grid,
        in_specs=[
            pl.BlockSpec((BB, D), lambda i: (i, 0)),
            pl.BlockSpec((M, D), lambda i: (0, 0)),
            pl.BlockSpec((M * C, D), lambda i: (0, 0)),
            pl.BlockSpec((1, M * C), lambda i: (0, 0)),
        ],
        out_specs=pl.BlockSpec((BB, C), lambda i: (i, 0)),
        out_shape=jax.ShapeDtypeStruct((B, C), jnp.float32),
    )(x, keys, Wf, bf)
